# baseline (speedup 1.0000x reference)
import functools

import jax
import jax.numpy as jnp
from jax import lax
from jax.experimental import pallas as pl
from jax.experimental.pallas import tpu as pltpu, tpu_sc as plsc


def _body(uh_ref, xu_ref, xi_ref, av_ref, cut_ref, cie_ref, st_ref, out_ref):
    I, P, Bn = xu_ref.shape
    cu = jnp.dot(cut_ref[...], uh_ref[...], preferred_element_type=jnp.float32)
    y = xu_ref[...] * cu[None, :, :] + xi_ref[...] * cie_ref[...]
    out = jnp.dot(st_ref[...], y.reshape(I * P, Bn),
                  preferred_element_type=jnp.float32)
    out_ref[...] = jnp.where(av_ref[...] != 0, out, jnp.float32(-1e20))


_ROWS = 800
_CHUNK = 32


def _sc_copy_body(x_hbm, out_hbm, buf, sem):
    wid = lax.axis_index("s") * 2 + lax.axis_index("c")
    base = wid * 512

    def body(r, c):
        pltpu.sync_copy(x_hbm.at[pl.ds(r * _CHUNK, _CHUNK), pl.ds(base, 512)], buf)
        pltpu.sync_copy(buf, out_hbm.at[pl.ds(r * _CHUNK, _CHUNK), pl.ds(base, 512)])
        return c

    lax.fori_loop(0, _ROWS // _CHUNK, body, 0)


def kernel(x_u, x_i, availability, user_onehot, coef_u, coef_i):
    B, I, P = x_u.shape
    U = coef_u.shape[0]
    IP = I * P

    xu_t = jnp.transpose(x_u, (1, 2, 0))
    xi_t = jnp.transpose(x_i, (1, 2, 0))
    uh_t = jnp.transpose(user_onehot, (1, 2, 0)).reshape(U, B)
    av_t = availability.T.astype(jnp.int8)
    cu_t = coef_u.T
    ci_e = coef_i[:, :, None]
    jj = jnp.arange(IP, dtype=jnp.int32)
    s_t = (jj[None, :] // P == jnp.arange(I, dtype=jnp.int32)[:, None]).astype(jnp.float32)

    xi_flat = xi_t.reshape(IP, B)
    mesh = plsc.VectorSubcoreMesh(core_axis_name="c", subcore_axis_name="s")
    sc_copy = functools.partial(
        pl.kernel,
        out_type=jax.ShapeDtypeStruct((_ROWS, B), jnp.float32),
        mesh=mesh,
        scratch_types=[
            pltpu.VMEM((_CHUNK, 512), jnp.float32),
            pltpu.SemaphoreType.DMA,
        ],
    )(_sc_copy_body)
    sc_out = sc_copy(xi_flat)

    Bn = 512
    out_t = pl.pallas_call(
        _body,
        grid=(B // Bn,),
        in_specs=[
            pl.BlockSpec((U, Bn), lambda i: (0, i)),
            pl.BlockSpec((I, P, Bn), lambda i: (0, 0, i)),
            pl.BlockSpec((I, P, Bn), lambda i: (0, 0, i)),
            pl.BlockSpec((I, Bn), lambda i: (0, i)),
            pl.BlockSpec((P, U), lambda i: (0, 0)),
            pl.BlockSpec((I, P, 1), lambda i: (0, 0, 0)),
            pl.BlockSpec((I, IP), lambda i: (0, 0)),
        ],
        out_specs=pl.BlockSpec((I, Bn), lambda i: (0, i)),
        out_shape=jax.ShapeDtypeStruct((I, B), jnp.float32),
    )(uh_t, xu_t, xi_t, av_t, cu_t, ci_e, s_t)
    return out_t.T + 0.0 * sc_out[0, 0]


# speedup vs baseline: 1.0512x; 1.0512x over previous
import functools

import jax
import jax.numpy as jnp
from jax import lax
from jax.experimental import pallas as pl
from jax.experimental.pallas import tpu as pltpu, tpu_sc as plsc

_BW = 512
_NX = 3
_NO = 4


def _sc_xi_body(x_hbm, w_hbm, out_hbm, xbuf, wbuf, obuf, xsems, wsems, osems):
    I = out_hbm.shape[0]
    P = 16
    wid = lax.axis_index("s") * 2 + lax.axis_index("c")
    base = pl.multiple_of(wid * _BW, _BW)

    def xsrc(c):
        r = pl.multiple_of(c * P, P)
        return x_hbm.at[pl.ds(r, P), pl.ds(base, _BW)]

    def wsrc(c):
        r = pl.multiple_of(c * P, P)
        return w_hbm.at[pl.ds(r, P), :]

    for c0 in range(_NX - 1):
        pltpu.async_copy(xsrc(c0), xbuf.at[c0], xsems.at[c0])
        pltpu.async_copy(wsrc(c0), wbuf.at[c0], wsems.at[c0])

    def body(c, carry):
        sx = lax.rem(c, _NX)
        so = lax.rem(c, _NO)
        nc = c + _NX - 1

        @pl.when(nc < I)
        def _():
            sn = lax.rem(nc, _NX)
            pltpu.async_copy(xsrc(nc), xbuf.at[sn], xsems.at[sn])
            pltpu.async_copy(wsrc(nc), wbuf.at[sn], wsems.at[sn])

        pltpu.make_async_copy(xsrc(c), xbuf.at[sx], xsems.at[sx]).wait()
        pltpu.make_async_copy(wsrc(c), wbuf.at[sx], wsems.at[sx]).wait()

        @pl.when(c >= _NO)
        def _():
            pltpu.make_async_copy(
                obuf.at[so], out_hbm.at[c - _NO, pl.ds(base, _BW)], osems.at[so]
            ).wait()

        ws = [wbuf[sx, p, :] for p in range(P)]

        def kbody(k, carry2):
            off = pl.multiple_of(k * 16, 16)
            acc = ws[0] * xbuf[sx, 0, pl.ds(off, 16)]
            for p in range(1, P):
                acc = acc + ws[p] * xbuf[sx, p, pl.ds(off, 16)]
            obuf[so, pl.ds(off, 16)] = acc
            return carry2

        lax.fori_loop(0, _BW // 16, kbody, 0)
        pltpu.async_copy(obuf.at[so], out_hbm.at[c, pl.ds(base, _BW)], osems.at[so])
        return carry

    lax.fori_loop(0, I, body, 0)
    for t in range(_NO):
        c = I - _NO + t
        pltpu.make_async_copy(
            obuf.at[c % _NO], out_hbm.at[c, pl.ds(base, _BW)], osems.at[c % _NO]
        ).wait()


def _body_xu(uh_ref, xu_ref, cut_ref, st_ref, t1_ref):
    I, P, Bn = xu_ref.shape
    cu = jnp.dot(cut_ref[...], uh_ref[...], preferred_element_type=jnp.float32)
    y = xu_ref[...] * cu[None, :, :]
    t1_ref[...] = jnp.dot(st_ref[...], y.reshape(I * P, Bn),
                          preferred_element_type=jnp.float32)


def _body_combine(t1_ref, pi_ref, av_ref, out_ref):
    out_ref[...] = jnp.where(av_ref[...] != 0, t1_ref[...] + pi_ref[...],
                             jnp.float32(-1e20))


def kernel(x_u, x_i, availability, user_onehot, coef_u, coef_i):
    B, I, P = x_u.shape
    U = coef_u.shape[0]
    IP = I * P

    xu_t = jnp.transpose(x_u, (1, 2, 0))
    xi_t = jnp.transpose(x_i, (1, 2, 0))
    uh_t = jnp.transpose(user_onehot, (1, 2, 0)).reshape(U, B)
    av_t = availability.T.astype(jnp.int8)
    cu_t = coef_u.T
    ci_b = jnp.broadcast_to(coef_i.reshape(IP, 1), (IP, 16))
    jj = jnp.arange(IP, dtype=jnp.int32)
    s_t = (jj[None, :] // P == jnp.arange(I, dtype=jnp.int32)[:, None]).astype(jnp.float32)

    xi_flat = xi_t.reshape(IP, B)
    mesh = plsc.VectorSubcoreMesh(core_axis_name="c", subcore_axis_name="s")
    sc_xi = functools.partial(
        pl.kernel,
        out_type=jax.ShapeDtypeStruct((I, B), jnp.float32),
        mesh=mesh,
        scratch_types=[
            pltpu.VMEM((_NX, P, _BW), jnp.float32),
            pltpu.VMEM((_NX, P, 16), jnp.float32),
            pltpu.VMEM((_NO, _BW), jnp.float32),
            pltpu.SemaphoreType.DMA((_NX,)),
            pltpu.SemaphoreType.DMA((_NX,)),
            pltpu.SemaphoreType.DMA((_NO,)),
        ],
    )(_sc_xi_body)
    partial = sc_xi(xi_flat, ci_b)

    Bn = 512
    t1 = pl.pallas_call(
        _body_xu,
        grid=(B // Bn,),
        in_specs=[
            pl.BlockSpec((U, Bn), lambda i: (0, i)),
            pl.BlockSpec((I, P, Bn), lambda i: (0, 0, i)),
            pl.BlockSpec((P, U), lambda i: (0, 0)),
            pl.BlockSpec((I, IP), lambda i: (0, 0)),
        ],
        out_specs=pl.BlockSpec((I, Bn), lambda i: (0, i)),
        out_shape=jax.ShapeDtypeStruct((I, B), jnp.float32),
    )(uh_t, xu_t, cu_t, s_t)

    Bc = 2048
    out_t = pl.pallas_call(
        _body_combine,
        grid=(B // Bc,),
        in_specs=[
            pl.BlockSpec((I, Bc), lambda i: (0, i)),
            pl.BlockSpec((I, Bc), lambda i: (0, i)),
            pl.BlockSpec((I, Bc), lambda i: (0, i)),
        ],
        out_specs=pl.BlockSpec((I, Bc), lambda i: (0, i)),
        out_shape=jax.ShapeDtypeStruct((I, B), jnp.float32),
    )(t1, partial, av_t)
    return out_t.T


# speedup vs baseline: 1.1216x; 1.0670x over previous
import functools

import jax
import jax.numpy as jnp
from jax import lax
from jax.experimental import pallas as pl
from jax.experimental.pallas import tpu as pltpu, tpu_sc as plsc

_BW = 512
_NX = 3
_NO = 4
_ISC = 50


def _sc_xi_body(x_hbm, w_hbm, out_hbm, xbuf, wbuf, obuf, xsems, wsems, osems):
    I = _ISC
    P = 16
    wid = lax.axis_index("s") * 2 + lax.axis_index("c")
    base = pl.multiple_of(wid * _BW, _BW)

    def xsrc(c):
        r = pl.multiple_of(c * P, P)
        return x_hbm.at[pl.ds(r, P), pl.ds(base, _BW)]

    def wsrc(c):
        r = pl.multiple_of(c * P, P)
        return w_hbm.at[pl.ds(r, P), :]

    for c0 in range(_NX - 1):
        pltpu.async_copy(xsrc(c0), xbuf.at[c0], xsems.at[c0])
        pltpu.async_copy(wsrc(c0), wbuf.at[c0], wsems.at[c0])

    def body(c, carry):
        sx = lax.rem(c, _NX)
        so = lax.rem(c, _NO)
        nc = c + _NX - 1

        @pl.when(nc < I)
        def _():
            sn = lax.rem(nc, _NX)
            pltpu.async_copy(xsrc(nc), xbuf.at[sn], xsems.at[sn])
            pltpu.async_copy(wsrc(nc), wbuf.at[sn], wsems.at[sn])

        pltpu.make_async_copy(xsrc(c), xbuf.at[sx], xsems.at[sx]).wait()
        pltpu.make_async_copy(wsrc(c), wbuf.at[sx], wsems.at[sx]).wait()

        @pl.when(c >= _NO)
        def _():
            pltpu.make_async_copy(
                obuf.at[so], out_hbm.at[c - _NO, pl.ds(base, _BW)], osems.at[so]
            ).wait()

        ws = [wbuf[sx, p, :] for p in range(P)]

        def kbody(k, carry2):
            off = pl.multiple_of(k * 16, 16)
            acc = ws[0] * xbuf[sx, 0, pl.ds(off, 16)]
            for p in range(1, P):
                acc = acc + ws[p] * xbuf[sx, p, pl.ds(off, 16)]
            obuf[so, pl.ds(off, 16)] = acc
            return carry2

        lax.fori_loop(0, _BW // 16, kbody, 0)
        pltpu.async_copy(obuf.at[so], out_hbm.at[c, pl.ds(base, _BW)], osems.at[so])
        return carry

    lax.fori_loop(0, I, body, 0)
    for t in range(_NO):
        c = I - _NO + t
        pltpu.make_async_copy(
            obuf.at[c % _NO], out_hbm.at[c, pl.ds(base, _BW)], osems.at[c % _NO]
        ).wait()


def _body_xu(uh_ref, xu_ref, xit_ref, cut_ref, cie_ref, st_ref, stt_ref, t1_ref):
    I, P, Bn = xu_ref.shape
    It = xit_ref.shape[0]
    cu = jnp.dot(cut_ref[...], uh_ref[...], preferred_element_type=jnp.float32)
    yu = xu_ref[...] * cu[None, :, :]
    yi = xit_ref[...] * cie_ref[...]
    t1 = jnp.dot(st_ref[...], yu.reshape(I * P, Bn),
                 preferred_element_type=jnp.float32)
    t1_ref[...] = t1 + jnp.dot(stt_ref[...], yi.reshape(It * P, Bn),
                               preferred_element_type=jnp.float32)


def _body_combine(t1_ref, pi_ref, av_ref, out_ref):
    I, Bc = t1_ref.shape
    row = lax.broadcasted_iota(jnp.int32, (I, Bc), 0)
    full = t1_ref[...] + jnp.where(row < _ISC, pi_ref[...], jnp.float32(0))
    out_ref[...] = jnp.where(av_ref[...] != 0, full, jnp.float32(-1e20))


def kernel(x_u, x_i, availability, user_onehot, coef_u, coef_i):
    B, I, P = x_u.shape
    U = coef_u.shape[0]
    IP = I * P
    It = I - _ISC

    xu_t = jnp.transpose(x_u, (1, 2, 0))
    xi_t = jnp.transpose(x_i, (1, 2, 0))
    uh_t = jnp.transpose(user_onehot, (1, 2, 0)).reshape(U, B)
    av_t = availability.T.astype(jnp.int8)
    cu_t = coef_u.T
    ci_b = jnp.broadcast_to(coef_i.reshape(IP, 1), (IP, 16))
    cie_tail = coef_i[_ISC:, :, None]
    jj = jnp.arange(IP, dtype=jnp.int32)
    ii = jnp.arange(I, dtype=jnp.int32)
    s_t = (jj[None, :] // P == ii[:, None]).astype(jnp.float32)
    jt = jnp.arange(It * P, dtype=jnp.int32)
    st_t = (_ISC + jt[None, :] // P == ii[:, None]).astype(jnp.float32)

    xi_flat = xi_t.reshape(IP, B)
    mesh = plsc.VectorSubcoreMesh(core_axis_name="c", subcore_axis_name="s")
    sc_xi = functools.partial(
        pl.kernel,
        out_type=jax.ShapeDtypeStruct((I, B), jnp.float32),
        mesh=mesh,
        scratch_types=[
            pltpu.VMEM((_NX, P, _BW), jnp.float32),
            pltpu.VMEM((_NX, P, 16), jnp.float32),
            pltpu.VMEM((_NO, _BW), jnp.float32),
            pltpu.SemaphoreType.DMA((_NX,)),
            pltpu.SemaphoreType.DMA((_NX,)),
            pltpu.SemaphoreType.DMA((_NO,)),
        ],
    )(_sc_xi_body)
    partial = sc_xi(xi_flat, ci_b)

    Bn = 512
    t1 = pl.pallas_call(
        _body_xu,
        grid=(B // Bn,),
        in_specs=[
            pl.BlockSpec((U, Bn), lambda i: (0, i)),
            pl.BlockSpec((I, P, Bn), lambda i: (0, 0, i)),
            pl.BlockSpec((It, P, Bn), lambda i: (1, 0, i)),
            pl.BlockSpec((P, U), lambda i: (0, 0)),
            pl.BlockSpec((It, P, 1), lambda i: (0, 0, 0)),
            pl.BlockSpec((I, IP), lambda i: (0, 0)),
            pl.BlockSpec((I, It * P), lambda i: (0, 0)),
        ],
        out_specs=pl.BlockSpec((I, Bn), lambda i: (0, i)),
        out_shape=jax.ShapeDtypeStruct((I, B), jnp.float32),
    )(uh_t, xu_t, xi_t, cu_t, cie_tail, s_t, st_t)

    Bc = 2048
    out_t = pl.pallas_call(
        _body_combine,
        grid=(B // Bc,),
        in_specs=[
            pl.BlockSpec((I, Bc), lambda i: (0, i)),
            pl.BlockSpec((I, Bc), lambda i: (0, i)),
            pl.BlockSpec((I, Bc), lambda i: (0, i)),
        ],
        out_specs=pl.BlockSpec((I, Bc), lambda i: (0, i)),
        out_shape=jax.ShapeDtypeStruct((I, B), jnp.float32),
    )(t1, partial, av_t)
    return out_t.T


# speedup vs baseline: 1.1241x; 1.0022x over previous
import functools

import jax
import jax.numpy as jnp
from jax import lax
from jax.experimental import pallas as pl
from jax.experimental.pallas import tpu as pltpu, tpu_sc as plsc

_BW = 512
_NX = 3
_NO = 4
_ISC = 50


def _sc_xi_body(x_hbm, w_hbm, out_hbm, xbuf, wbuf, obuf, xsems, wsems, osems):
    I = _ISC
    P = 16
    wid = lax.axis_index("s") * 2 + lax.axis_index("c")
    base = pl.multiple_of(wid * _BW, _BW)

    def xsrc(c):
        r = pl.multiple_of(c * P, P)
        return x_hbm.at[pl.ds(r, P), pl.ds(base, _BW)]

    def wsrc(c):
        r = pl.multiple_of(c * P, P)
        return w_hbm.at[pl.ds(r, P), :]

    for c0 in range(_NX - 1):
        pltpu.async_copy(xsrc(c0), xbuf.at[c0], xsems.at[c0])
        pltpu.async_copy(wsrc(c0), wbuf.at[c0], wsems.at[c0])

    def body(c, carry):
        sx = lax.rem(c, _NX)
        so = lax.rem(c, _NO)
        nc = c + _NX - 1

        @pl.when(nc < I)
        def _():
            sn = lax.rem(nc, _NX)
            pltpu.async_copy(xsrc(nc), xbuf.at[sn], xsems.at[sn])
            pltpu.async_copy(wsrc(nc), wbuf.at[sn], wsems.at[sn])

        pltpu.make_async_copy(xsrc(c), xbuf.at[sx], xsems.at[sx]).wait()
        pltpu.make_async_copy(wsrc(c), wbuf.at[sx], wsems.at[sx]).wait()

        @pl.when(c >= _NO)
        def _():
            pltpu.make_async_copy(
                obuf.at[so], out_hbm.at[c - _NO, pl.ds(base, _BW)], osems.at[so]
            ).wait()

        ws = [wbuf[sx, p, :] for p in range(P)]

        def kbody(k, carry2):
            for u in range(2):
                off = pl.multiple_of(k * 32 + u * 16, 16)
                t = [ws[p] * xbuf[sx, p, pl.ds(off, 16)] for p in range(P)]
                while len(t) > 1:
                    t = [t[2 * j] + t[2 * j + 1] for j in range(len(t) // 2)]
                obuf[so, pl.ds(off, 16)] = t[0]
            return carry2

        lax.fori_loop(0, _BW // 32, kbody, 0)
        pltpu.async_copy(obuf.at[so], out_hbm.at[c, pl.ds(base, _BW)], osems.at[so])
        return carry

    lax.fori_loop(0, I, body, 0)
    for t in range(_NO):
        c = I - _NO + t
        pltpu.make_async_copy(
            obuf.at[c % _NO], out_hbm.at[c, pl.ds(base, _BW)], osems.at[c % _NO]
        ).wait()


def _body_xu(uh_ref, xu_ref, xit_ref, cut_ref, cie_ref, st_ref, stt_ref, t1_ref):
    I, P, Bn = xu_ref.shape
    It = xit_ref.shape[0]
    cu = jnp.dot(cut_ref[...], uh_ref[...], preferred_element_type=jnp.float32)
    yu = xu_ref[...] * cu[None, :, :]
    yi = xit_ref[...] * cie_ref[...]
    t1 = jnp.dot(st_ref[...], yu.reshape(I * P, Bn),
                 preferred_element_type=jnp.float32)
    t1_ref[...] = t1 + jnp.dot(stt_ref[...], yi.reshape(It * P, Bn),
                               preferred_element_type=jnp.float32)


def _body_combine(t1_ref, pi_ref, av_ref, out_ref):
    I, Bc = t1_ref.shape
    row = lax.broadcasted_iota(jnp.int32, (I, Bc), 0)
    full = t1_ref[...] + jnp.where(row < _ISC, pi_ref[...], jnp.float32(0))
    out_ref[...] = jnp.where(av_ref[...] != 0, full, jnp.float32(-1e20))


def kernel(x_u, x_i, availability, user_onehot, coef_u, coef_i):
    B, I, P = x_u.shape
    U = coef_u.shape[0]
    IP = I * P
    It = I - _ISC

    xu_t = jnp.transpose(x_u, (1, 2, 0))
    xi_t = jnp.transpose(x_i, (1, 2, 0))
    uh_t = jnp.transpose(user_onehot, (1, 2, 0)).reshape(U, B)
    av_t = availability.T.astype(jnp.int8)
    cu_t = coef_u.T
    ci_b = jnp.broadcast_to(coef_i.reshape(IP, 1), (IP, 16))
    cie_tail = coef_i[_ISC:, :, None]
    jj = jnp.arange(IP, dtype=jnp.int32)
    ii = jnp.arange(I, dtype=jnp.int32)
    s_t = (jj[None, :] // P == ii[:, None]).astype(jnp.float32)
    jt = jnp.arange(It * P, dtype=jnp.int32)
    st_t = (_ISC + jt[None, :] // P == ii[:, None]).astype(jnp.float32)

    xi_flat = xi_t.reshape(IP, B)
    mesh = plsc.VectorSubcoreMesh(core_axis_name="c", subcore_axis_name="s")
    sc_xi = functools.partial(
        pl.kernel,
        out_type=jax.ShapeDtypeStruct((I, B), jnp.float32),
        mesh=mesh,
        scratch_types=[
            pltpu.VMEM((_NX, P, _BW), jnp.float32),
            pltpu.VMEM((_NX, P, 16), jnp.float32),
            pltpu.VMEM((_NO, _BW), jnp.float32),
            pltpu.SemaphoreType.DMA((_NX,)),
            pltpu.SemaphoreType.DMA((_NX,)),
            pltpu.SemaphoreType.DMA((_NO,)),
        ],
    )(_sc_xi_body)
    partial = sc_xi(xi_flat, ci_b)

    Bn = 512
    t1 = pl.pallas_call(
        _body_xu,
        grid=(B // Bn,),
        in_specs=[
            pl.BlockSpec((U, Bn), lambda i: (0, i)),
            pl.BlockSpec((I, P, Bn), lambda i: (0, 0, i)),
            pl.BlockSpec((It, P, Bn), lambda i: (1, 0, i)),
            pl.BlockSpec((P, U), lambda i: (0, 0)),
            pl.BlockSpec((It, P, 1), lambda i: (0, 0, 0)),
            pl.BlockSpec((I, IP), lambda i: (0, 0)),
            pl.BlockSpec((I, It * P), lambda i: (0, 0)),
        ],
        out_specs=pl.BlockSpec((I, Bn), lambda i: (0, i)),
        out_shape=jax.ShapeDtypeStruct((I, B), jnp.float32),
    )(uh_t, xu_t, xi_t, cu_t, cie_tail, s_t, st_t)

    Bc = 2048
    out_t = pl.pallas_call(
        _body_combine,
        grid=(B // Bc,),
        in_specs=[
            pl.BlockSpec((I, Bc), lambda i: (0, i)),
            pl.BlockSpec((I, Bc), lambda i: (0, i)),
            pl.BlockSpec((I, Bc), lambda i: (0, i)),
        ],
        out_specs=pl.BlockSpec((I, Bc), lambda i: (0, i)),
        out_shape=jax.ShapeDtypeStruct((I, B), jnp.float32),
    )(t1, partial, av_t)
    return out_t.T


# speedup vs baseline: 1.2015x; 1.0689x over previous
import functools

import jax
import jax.numpy as jnp
from jax import lax
from jax.experimental import pallas as pl
from jax.experimental.pallas import tpu as pltpu, tpu_sc as plsc

_CG = 8
_IG = 4
_CW = 16384 // _CG
_NX = 2
_NO = 2
_ISC = 80
_IPW = _ISC // _IG


def _sc_xi_body(x_hbm, w_hbm, out_hbm, xbuf, wbuf, obuf, xsems, wsems, osems):
    P = 16
    wid = lax.axis_index("s") * 2 + lax.axis_index("c")
    colg = lax.rem(wid, _CG)
    itemg = wid // _CG
    base = pl.multiple_of(colg * _CW, _CW)
    i0 = itemg * _IPW

    def xsrc(k):
        r = pl.multiple_of((i0 + k) * P, P)
        return x_hbm.at[pl.ds(r, P), pl.ds(base, _CW)]

    def wsrc(k):
        r = pl.multiple_of((i0 + k) * P, P)
        return w_hbm.at[pl.ds(r, P), :]

    for c0 in range(_NX - 1):
        pltpu.async_copy(xsrc(c0), xbuf.at[c0], xsems.at[c0])
        pltpu.async_copy(wsrc(c0), wbuf.at[c0], wsems.at[c0])

    def body(c, carry):
        sx = lax.rem(c, _NX)
        so = lax.rem(c, _NO)
        nc = c + _NX - 1

        @pl.when(nc < _IPW)
        def _():
            sn = lax.rem(nc, _NX)
            pltpu.async_copy(xsrc(nc), xbuf.at[sn], xsems.at[sn])
            pltpu.async_copy(wsrc(nc), wbuf.at[sn], wsems.at[sn])

        pltpu.make_async_copy(xsrc(c), xbuf.at[sx], xsems.at[sx]).wait()
        pltpu.make_async_copy(wsrc(c), wbuf.at[sx], wsems.at[sx]).wait()

        @pl.when(c >= _NO)
        def _():
            pltpu.make_async_copy(
                obuf.at[so], out_hbm.at[i0 + c - _NO, pl.ds(base, _CW)],
                osems.at[so],
            ).wait()

        ws = [wbuf[sx, p, :] for p in range(P)]

        def kbody(k, carry2):
            for u in range(2):
                off = pl.multiple_of(k * 32 + u * 16, 16)
                t = [ws[p] * xbuf[sx, p, pl.ds(off, 16)] for p in range(P)]
                while len(t) > 1:
                    t = [t[2 * j] + t[2 * j + 1] for j in range(len(t) // 2)]
                obuf[so, pl.ds(off, 16)] = t[0]
            return carry2

        lax.fori_loop(0, _CW // 32, kbody, 0)
        pltpu.async_copy(
            obuf.at[so], out_hbm.at[i0 + c, pl.ds(base, _CW)], osems.at[so]
        )
        return carry

    lax.fori_loop(0, _IPW, body, 0)
    for t in range(_NO):
        c = _IPW - _NO + t
        pltpu.make_async_copy(
            obuf.at[c % _NO], out_hbm.at[i0 + c, pl.ds(base, _CW)],
            osems.at[c % _NO],
        ).wait()


def _body_xu(uh_ref, xu_ref, xit_ref, cut_ref, cie_ref, st_ref, stt_ref, t1_ref):
    I, P, Bn = xu_ref.shape
    It = xit_ref.shape[0]
    cu = jnp.dot(cut_ref[...], uh_ref[...], preferred_element_type=jnp.float32)
    yu = xu_ref[...] * cu[None, :, :]
    yi = xit_ref[...] * cie_ref[...]
    t1 = jnp.dot(st_ref[...], yu.reshape(I * P, Bn),
                 preferred_element_type=jnp.float32)
    t1_ref[...] = t1 + jnp.dot(stt_ref[...], yi.reshape(It * P, Bn),
                               preferred_element_type=jnp.float32)


def _body_combine(t1_ref, pi_ref, av_ref, out_ref):
    I, Bc = t1_ref.shape
    row = lax.broadcasted_iota(jnp.int32, (I, Bc), 0)
    full = t1_ref[...] + jnp.where(row < _ISC, pi_ref[...], jnp.float32(0))
    out_ref[...] = jnp.where(av_ref[...] != 0, full, jnp.float32(-1e20))


def kernel(x_u, x_i, availability, user_onehot, coef_u, coef_i):
    B, I, P = x_u.shape
    U = coef_u.shape[0]
    IP = I * P
    It = I - _ISC

    xu_t = jnp.transpose(x_u, (1, 2, 0))
    xi_t = jnp.transpose(x_i, (1, 2, 0))
    uh_t = jnp.transpose(user_onehot, (1, 2, 0)).reshape(U, B)
    av_t = availability.T.astype(jnp.int8)
    cu_t = coef_u.T
    ci_b = jnp.broadcast_to(coef_i.reshape(IP, 1), (IP, 16))
    cie_tail = coef_i[_ISC:, :, None]
    jj = jnp.arange(IP, dtype=jnp.int32)
    ii = jnp.arange(I, dtype=jnp.int32)
    s_t = (jj[None, :] // P == ii[:, None]).astype(jnp.float32)
    jt = jnp.arange(It * P, dtype=jnp.int32)
    st_t = (_ISC + jt[None, :] // P == ii[:, None]).astype(jnp.float32)

    xi_flat = xi_t.reshape(IP, B)
    mesh = plsc.VectorSubcoreMesh(core_axis_name="c", subcore_axis_name="s")
    sc_xi = functools.partial(
        pl.kernel,
        out_type=jax.ShapeDtypeStruct((I, B), jnp.float32),
        mesh=mesh,
        scratch_types=[
            pltpu.VMEM((_NX, P, _CW), jnp.float32),
            pltpu.VMEM((_NX, P, 16), jnp.float32),
            pltpu.VMEM((_NO, _CW), jnp.float32),
            pltpu.SemaphoreType.DMA((_NX,)),
            pltpu.SemaphoreType.DMA((_NX,)),
            pltpu.SemaphoreType.DMA((_NO,)),
        ],
    )(_sc_xi_body)
    partial = sc_xi(xi_flat, ci_b)

    Bn = 512
    t1 = pl.pallas_call(
        _body_xu,
        grid=(B // Bn,),
        in_specs=[
            pl.BlockSpec((U, Bn), lambda i: (0, i)),
            pl.BlockSpec((I, P, Bn), lambda i: (0, 0, i)),
            pl.BlockSpec((It, P, Bn), lambda i: (_ISC // (100 - _ISC), 0, i)),
            pl.BlockSpec((P, U), lambda i: (0, 0)),
            pl.BlockSpec((It, P, 1), lambda i: (0, 0, 0)),
            pl.BlockSpec((I, IP), lambda i: (0, 0)),
            pl.BlockSpec((I, It * P), lambda i: (0, 0)),
        ],
        out_specs=pl.BlockSpec((I, Bn), lambda i: (0, i)),
        out_shape=jax.ShapeDtypeStruct((I, B), jnp.float32),
    )(uh_t, xu_t, xi_t, cu_t, cie_tail, s_t, st_t)

    Bc = 2048
    out_t = pl.pallas_call(
        _body_combine,
        grid=(B // Bc,),
        in_specs=[
            pl.BlockSpec((I, Bc), lambda i: (0, i)),
            pl.BlockSpec((I, Bc), lambda i: (0, i)),
            pl.BlockSpec((I, Bc), lambda i: (0, i)),
        ],
        out_specs=pl.BlockSpec((I, Bc), lambda i: (0, i)),
        out_shape=jax.ShapeDtypeStruct((I, B), jnp.float32),
    )(t1, partial, av_t)
    return out_t.T


# speedup vs baseline: 1.6083x; 1.3386x over previous
import jax
import jax.numpy as jnp
from jax import lax
from jax.experimental import pallas as pl


def _body(uh_ref, xu_ref, xi_ref, av_ref, cut_ref, cie_ref, s_ref, out_ref):
    I, P, Bn = xu_ref.shape
    cu = jnp.dot(cut_ref[...], uh_ref[...], preferred_element_type=jnp.float32)
    yu = xu_ref[...] * cu[None, :, :]
    yi = xi_ref[...] * cie_ref[...]
    t = jnp.dot(s_ref[...], (yu + yi).reshape(I * P, Bn),
                preferred_element_type=jnp.float32)
    out_ref[...] = jnp.where(av_ref[...] != 0, t, jnp.float32(-1e20))


def kernel(x_u, x_i, availability, user_onehot, coef_u, coef_i):
    B, I, P = x_u.shape
    U = coef_u.shape[0]
    IP = I * P

    xu_t = jnp.transpose(x_u, (1, 2, 0))
    xi_t = jnp.transpose(x_i, (1, 2, 0))
    uh_t = jnp.transpose(user_onehot, (1, 2, 0)).reshape(U, B)
    av_t = availability.T.astype(jnp.int8)
    cu_t = coef_u.T
    cie = coef_i[:, :, None]
    jj = jnp.arange(IP, dtype=jnp.int32)
    ii = jnp.arange(I, dtype=jnp.int32)
    s_t = (jj[None, :] // P == ii[:, None]).astype(jnp.float32)

    Bn = 512
    out_t = pl.pallas_call(
        _body,
        grid=(B // Bn,),
        in_specs=[
            pl.BlockSpec((U, Bn), lambda i: (0, i)),
            pl.BlockSpec((I, P, Bn), lambda i: (0, 0, i)),
            pl.BlockSpec((I, P, Bn), lambda i: (0, 0, i)),
            pl.BlockSpec((I, Bn), lambda i: (0, i)),
            pl.BlockSpec((P, U), lambda i: (0, 0)),
            pl.BlockSpec((I, P, 1), lambda i: (0, 0, 0)),
            pl.BlockSpec((I, IP), lambda i: (0, 0)),
        ],
        out_specs=pl.BlockSpec((I, Bn), lambda i: (0, i)),
        out_shape=jax.ShapeDtypeStruct((I, B), jnp.float32),
    )(uh_t, xu_t, xi_t, av_t, cu_t, cie, s_t)
    return out_t.T
